# Initial kernel scaffold; baseline (speedup 1.0000x reference)
import jax
import jax.numpy as jnp
from jax.experimental import pallas as pl


def kernel(x, edge_index, batch, W1, a_src1, a_dst1, b1, W2, a_src2, a_dst2, b2, W3, a_src3, a_dst3, b3, Wl, bl):
    raise NotImplementedError("write your pallas kernel here")


# speedup vs baseline: 26.8776x; 26.8776x over previous
import functools

import jax
import jax.numpy as jnp
from jax import lax
from jax.experimental import pallas as pl
from jax.experimental.pallas import tpu as pltpu
from jax.experimental.pallas import tpu_sc as plsc

N = 10000
E = 320000
D = 128
HID = 32
HEADS = 8
OUT = 10
G = 64

Bn = 1000
NB = N // Bn
CH = 64
EPAD = 321536
DUMMY_DST = 10200
EPT = EPAD // 16
EPT3 = EPAD // 32
NPAD = 10240
DEN0 = NPAD
NDEN = 512
NACC = NPAD + NDEN
RPT = NPAD // 16

_f32 = jnp.float32


def _leaky(x):
    return jnp.maximum(x, 0.2 * x)


def _elu(x):
    return jnp.where(x > 0, x, jnp.exp(jnp.minimum(x, 0.0)) - 1.0)


def _head_expand():
    lane = lax.broadcasted_iota(jnp.int32, (4, 128), 1) // 32
    head = lax.broadcasted_iota(jnp.int32, (4, 128), 0)
    return (lane == head).astype(_f32)


def _track_max(i, asb, adb, acc_ref, m_ref):
    smax = jnp.max(asb, axis=0, keepdims=True)
    dmax = jnp.max(adb, axis=0, keepdims=True)
    nh = asb.shape[1]

    @pl.when(i == 0)
    def _():
        acc_ref[0:1, 0:nh] = smax
        acc_ref[1:2, 0:nh] = dmax

    @pl.when(i > 0)
    def _():
        acc_ref[0:1, 0:nh] = jnp.maximum(acc_ref[0:1, 0:nh], smax)
        acc_ref[1:2, 0:nh] = jnp.maximum(acc_ref[1:2, 0:nh], dmax)

    @pl.when(i == NB - 1)
    def _():
        m = _leaky(acc_ref[0:1, 0:nh] + acc_ref[1:2, 0:nh])
        mpad = jnp.concatenate([m, jnp.zeros((1, 128 - nh), _f32)], axis=1)
        rowi = lax.broadcasted_iota(jnp.int32, (8, 128), 0)
        msplat = (rowi == 4).astype(_f32) * mpad
        for k in range(nh):
            msplat = msplat + (rowi == k).astype(_f32) * m[0:1, k:k + 1]
        m_ref[...] = msplat[None]


def _a1_body(x_ref, w_ref, wsd_ref, h_ref, ast_ref, adt_ref, m_ref, acc_ref):
    i = pl.program_id(1)
    xb = x_ref[...]
    hb = jnp.dot(xb, w_ref[0], preferred_element_type=_f32)
    h_ref[...] = hb
    sd = jnp.dot(xb, wsd_ref[0], preferred_element_type=_f32)
    ast_ref[0] = sd[:, 0:4]
    adt_ref[0] = sd[:, 4:8]
    _track_max(i, sd[:, 0:4], sd[:, 4:8], acc_ref, m_ref)


def _x_from_prev(acc_ref, den_ref, hprev_ref, asprev_ref, adprev_ref,
                 mprev_ref, b_ref):
    expand = _head_expand()
    parts = []
    for c in range(2):
        hb = hprev_ref[c]
        e = _leaky(asprev_ref[c] + adprev_ref[c])
        ws = jnp.exp(e - mprev_ref[c, 4:5, 0:4])
        den = den_ref[c] + ws
        repw = jnp.dot(ws, expand, preferred_element_type=_f32)
        repr_ = jnp.dot(1.0 / den, expand, preferred_element_type=_f32)
        xc = (acc_ref[c] + repw * hb) * repr_ + b_ref[c:c + 1, :]
        parts.append(_elu(xc))
    return jnp.concatenate(parts, axis=1)


def _a2_body(acc_ref, den_ref, hprev_ref, asprev_ref, adprev_ref, mprev_ref,
             b_ref, w_ref, wsd_ref, h_ref, ast_ref, adt_ref, m_ref, macc_ref):
    i = pl.program_id(1)
    x2 = _x_from_prev(acc_ref, den_ref, hprev_ref, asprev_ref, adprev_ref,
                      mprev_ref, b_ref)
    hb = jnp.dot(x2, w_ref[0], preferred_element_type=_f32)
    h_ref[...] = hb
    sd = jnp.dot(x2, wsd_ref[0], preferred_element_type=_f32)
    ast_ref[0] = sd[:, 0:4]
    adt_ref[0] = sd[:, 4:8]
    _track_max(i, sd[:, 0:4], sd[:, 4:8], macc_ref, m_ref)


def _track_max_one(i, asb, adb, acc_ref, m_ref):
    smax = jnp.max(asb, axis=0, keepdims=True)
    dmax = jnp.max(adb, axis=0, keepdims=True)

    @pl.when(i == 0)
    def _():
        acc_ref[0:1, 0:1] = smax
        acc_ref[1:2, 0:1] = dmax

    @pl.when(i > 0)
    def _():
        acc_ref[0:1, 0:1] = jnp.maximum(acc_ref[0:1, 0:1], smax)
        acc_ref[1:2, 0:1] = jnp.maximum(acc_ref[1:2, 0:1], dmax)

    @pl.when(i == NB - 1)
    def _():
        m = _leaky(acc_ref[0:1, 0:1] + acc_ref[1:2, 0:1])
        mpad = jnp.concatenate([m, jnp.zeros((1, 127), _f32)], axis=1)
        rowi = lax.broadcasted_iota(jnp.int32, (8, 128), 0)
        msplat = ((rowi == 4).astype(_f32) * mpad
                  + (rowi == 0).astype(_f32) * m[0:1, 0:1])
        m_ref[...] = msplat[None]


def _a3_body(acc_ref, den_ref, hprev_ref, asprev_ref, adprev_ref, mprev_ref,
             b_ref, w_ref, wd_ref, h_ref, adt_ref, m_ref, macc_ref):
    i = pl.program_id(0)
    x3 = _x_from_prev(acc_ref, den_ref, hprev_ref, asprev_ref, adprev_ref,
                      mprev_ref, b_ref)
    hb = jnp.dot(x3, w_ref[...], preferred_element_type=_f32)
    h_ref[...] = hb
    adb = jnp.dot(x3, wd_ref[...], preferred_element_type=_f32)
    adt_ref[...] = adb
    _track_max_one(i, hb[:, 32:33], adb, macc_ref, m_ref)


def _pool_body(acc_ref, hprev_ref, adprev_ref, mprev_ref, b3_ref, batch_ref,
               wl_ref, bl_ref, out_ref, sum_ref, cnt_ref):
    i = pl.program_id(0)
    hb = hprev_ref[...]
    e = _leaky(hb[:, 32:33] + adprev_ref[...])
    ws = jnp.exp(e - mprev_ref[0, 4:5, 0:1])
    den = acc_ref[0][:, 32:33] + acc_ref[1][:, 32:33] + ws
    num = acc_ref[0][:, 0:32] + acc_ref[1][:, 0:32] + ws * hb[:, 0:32]
    r = jnp.maximum(num / den + b3_ref[...], 0.0)

    bb = batch_ref[...]
    onehot = (lax.broadcasted_iota(jnp.int32, (Bn, G), 1)
              == bb).astype(_f32)
    dn = (((0,), (0,)), ((), ()))
    psum = lax.dot_general(onehot, r, dn, preferred_element_type=_f32)
    pcnt = lax.dot_general(onehot, jnp.ones((Bn, 32), _f32), dn,
                           preferred_element_type=_f32)

    @pl.when(i == 0)
    def _():
        sum_ref[...] = psum
        cnt_ref[...] = pcnt

    @pl.when(i > 0)
    def _():
        sum_ref[...] = sum_ref[...] + psum
        cnt_ref[...] = cnt_ref[...] + pcnt

    @pl.when(i == NB - 1)
    def _():
        pooled = sum_ref[...] / jnp.maximum(cnt_ref[...], 1.0)
        out_ref[...] = (jnp.dot(pooled, wl_ref[...],
                                preferred_element_type=_f32) + bl_ref[...])


def _a1_call(x, w, wsd):
    return pl.pallas_call(
        _a1_body,
        grid=(2, NB),
        in_specs=[
            pl.BlockSpec((Bn, 128), lambda j, i: (i, 0)),
            pl.BlockSpec((1, 128, 128), lambda j, i: (j, 0, 0)),
            pl.BlockSpec((1, 128, 8), lambda j, i: (j, 0, 0)),
        ],
        out_specs=[
            pl.BlockSpec((Bn, 128), lambda j, i: (j * NB + i, 0)),
            pl.BlockSpec((1, Bn, 4), lambda j, i: (j, i, 0)),
            pl.BlockSpec((1, Bn, 4), lambda j, i: (j, i, 0)),
            pl.BlockSpec((1, 8, 128), lambda j, i: (j, 0, 0)),
        ],
        out_shape=[
            jax.ShapeDtypeStruct((2 * N, 128), _f32),
            jax.ShapeDtypeStruct((2, N, 4), _f32),
            jax.ShapeDtypeStruct((2, N, 4), _f32),
            jax.ShapeDtypeStruct((2, 8, 128), _f32),
        ],
        scratch_shapes=[pltpu.VMEM((8, 128), _f32)],
    )(x, w, wsd)


def _a2_call(acc, den, hprev, asprev, adprev, mprev, b, w, wsd):
    return pl.pallas_call(
        _a2_body,
        grid=(2, NB),
        in_specs=[
            pl.BlockSpec((2, Bn, 128), lambda j, i: (0, i, 0)),
            pl.BlockSpec((2, Bn, 4), lambda j, i: (0, i, 0)),
            pl.BlockSpec((2, Bn, 128), lambda j, i: (0, i, 0)),
            pl.BlockSpec((2, Bn, 4), lambda j, i: (0, i, 0)),
            pl.BlockSpec((2, Bn, 4), lambda j, i: (0, i, 0)),
            pl.BlockSpec((2, 8, 128), lambda j, i: (0, 0, 0)),
            pl.BlockSpec((2, 128), lambda j, i: (0, 0)),
            pl.BlockSpec((1, 256, 128), lambda j, i: (j, 0, 0)),
            pl.BlockSpec((1, 256, 8), lambda j, i: (j, 0, 0)),
        ],
        out_specs=[
            pl.BlockSpec((Bn, 128), lambda j, i: (j * NB + i, 0)),
            pl.BlockSpec((1, Bn, 4), lambda j, i: (j, i, 0)),
            pl.BlockSpec((1, Bn, 4), lambda j, i: (j, i, 0)),
            pl.BlockSpec((1, 8, 128), lambda j, i: (j, 0, 0)),
        ],
        out_shape=[
            jax.ShapeDtypeStruct((2 * N, 128), _f32),
            jax.ShapeDtypeStruct((2, N, 4), _f32),
            jax.ShapeDtypeStruct((2, N, 4), _f32),
            jax.ShapeDtypeStruct((2, 8, 128), _f32),
        ],
        scratch_shapes=[pltpu.VMEM((8, 128), _f32)],
    )(acc, den, hprev, asprev, adprev, mprev, b, w, wsd)


def _a3_call(acc, den, hprev, asprev, adprev, mprev, b, w, wd):
    return pl.pallas_call(
        _a3_body,
        grid=(NB,),
        in_specs=[
            pl.BlockSpec((2, Bn, 128), lambda i: (0, i, 0)),
            pl.BlockSpec((2, Bn, 4), lambda i: (0, i, 0)),
            pl.BlockSpec((2, Bn, 128), lambda i: (0, i, 0)),
            pl.BlockSpec((2, Bn, 4), lambda i: (0, i, 0)),
            pl.BlockSpec((2, Bn, 4), lambda i: (0, i, 0)),
            pl.BlockSpec((2, 8, 128), lambda i: (0, 0, 0)),
            pl.BlockSpec((2, 128), lambda i: (0, 0)),
            pl.BlockSpec((256, 128), lambda i: (0, 0)),
            pl.BlockSpec((256, 1), lambda i: (0, 0)),
        ],
        out_specs=[
            pl.BlockSpec((Bn, 128), lambda i: (i, 0)),
            pl.BlockSpec((Bn, 1), lambda i: (i, 0)),
            pl.BlockSpec((1, 8, 128), lambda i: (0, 0, 0)),
        ],
        out_shape=[
            jax.ShapeDtypeStruct((N, 128), _f32),
            jax.ShapeDtypeStruct((N, 1), _f32),
            jax.ShapeDtypeStruct((1, 8, 128), _f32),
        ],
        scratch_shapes=[pltpu.VMEM((8, 128), _f32)],
    )(acc, den, hprev, asprev, adprev, mprev, b, w, wd)


def _pool_call(acc3, h3, adt3, m3, b3, batch, wl, bl):
    return pl.pallas_call(
        _pool_body,
        grid=(NB,),
        in_specs=[
            pl.BlockSpec((2, Bn, 128), lambda i: (0, i, 0)),
            pl.BlockSpec((Bn, 128), lambda i: (i, 0)),
            pl.BlockSpec((Bn, 1), lambda i: (i, 0)),
            pl.BlockSpec((1, 8, 128), lambda i: (0, 0, 0)),
            pl.BlockSpec((1, 32), lambda i: (0, 0)),
            pl.BlockSpec((Bn, 1), lambda i: (i, 0)),
            pl.BlockSpec((32, OUT), lambda i: (0, 0)),
            pl.BlockSpec((1, OUT), lambda i: (0, 0)),
        ],
        out_specs=pl.BlockSpec((G, OUT), lambda i: (0, 0)),
        out_shape=jax.ShapeDtypeStruct((G, OUT), _f32),
        scratch_shapes=[pltpu.VMEM((G, 32), _f32), pltpu.VMEM((G, 32), _f32)],
    )(acc3, h3, adt3, m3, b3, batch, wl, bl)


def _iota16():
    return lax.broadcasted_iota(jnp.int32, (16,), 0)


def _zero_rows(buf, nrows):
    z = jnp.zeros((16,), _f32)

    def body(i, _):
        for q in range(8):
            buf[i, pl.ds(q * 16, 16)] = z
        return 0

    lax.fori_loop(0, nrows, body, 0)


@functools.cache
def _edge12_kernel():
    return pl.kernel(
        _edge12_body,
        out_type=jax.ShapeDtypeStruct((2, NACC, 128), _f32),
        mesh=plsc.VectorSubcoreMesh(core_axis_name="c", subcore_axis_name="s"),
        compiler_params=pltpu.CompilerParams(needs_layout_passes=False),
        scratch_types=[
            pltpu.VMEM_SHARED((NACC, 128), _f32),
            pltpu.VMEM_SHARED((320, 128), _f32),
            pltpu.VMEM_SHARED((320, 128), _f32),
            pltpu.VMEM((8, 128), _f32),
            pltpu.VMEM((CH,), jnp.int32),
            pltpu.VMEM((CH,), jnp.int32),
            pltpu.VMEM((CH,), jnp.int32),
            pltpu.VMEM((CH,), jnp.int32),
            pltpu.VMEM((CH, 128), _f32),
            pltpu.VMEM((CH, 128), _f32),
            pltpu.VMEM((CH, 128), _f32),
            pltpu.VMEM((8, 128), _f32),
            pltpu.SemaphoreType.DMA,
        ],
    )


def _edge12_body(hrows, srcf, dstf, aspk, adpk, mp, out, acc_sh, as_sh, ad_sh,
                 m_vec, srcbuf, dstbuf, asid, adid, gbuf, rowbuf, denbuf,
                 estash, sem):
    c = lax.axis_index("c")
    s = lax.axis_index("s")
    r0 = s * RPT
    d0 = DEN0 + s * (NDEN // 16)

    @pl.when(s == 0)
    def _():
        pltpu.sync_copy(aspk.at[c], as_sh)
        pltpu.sync_copy(adpk.at[c], ad_sh)

    _zero_rows(rowbuf, CH)
    _zero_rows(denbuf, CH)
    for q in range(10):
        pltpu.sync_copy(rowbuf, acc_sh.at[pl.ds(r0 + q * CH, CH)])
    pltpu.sync_copy(rowbuf.at[pl.ds(0, 32)], acc_sh.at[pl.ds(d0, 32)])
    pltpu.sync_copy(mp.at[c], m_vec)
    plsc.subcore_barrier()

    c_n = c * N
    zero16 = jnp.zeros((16,), _f32)

    def chunk(k, _):
        base = s * EPT + k * CH
        pltpu.sync_copy(srcf.at[pl.ds(base, CH)], srcbuf)
        pltpu.sync_copy(dstf.at[pl.ds(base, CH)], dstbuf)

        def prep(t, _):
            sl = pl.ds(t * 16, 16)
            asid[sl] = lax.shift_right_logical(srcbuf[sl], 5)
            adid[sl] = lax.shift_right_logical(dstbuf[sl], 5)
            return 0

        lax.fori_loop(0, CH // 16, prep, 0)

        pltpu.async_copy(as_sh.at[asid], gbuf, sem).wait()

        def pa(t, _):
            sl = pl.ds(t * 16, 16)
            ridx = _iota16() + t * 16
            lbase = (srcbuf[sl] & 31) * 4
            for h in range(4):
                estash[h, sl] = plsc.load_gather(gbuf, [ridx, lbase + h])
            srcbuf[sl] = srcbuf[sl] + c_n
            return 0

        lax.fori_loop(0, CH // 16, pa, 0)

        pltpu.async_copy(ad_sh.at[adid], gbuf, sem).wait()

        def pb(t, _):
            sl = pl.ds(t * 16, 16)
            ridx = _iota16() + t * 16
            lbase = (dstbuf[sl] & 31) * 4
            for h in range(4):
                adv = plsc.load_gather(gbuf, [ridx, lbase + h])
                e = _leaky(estash[h, sl] + adv) - m_vec[h, pl.ds(0, 16)]
                estash[4 + h, sl] = jnp.exp(e)
            return 0

        lax.fori_loop(0, CH // 16, pb, 0)

        pltpu.async_copy(hrows.at[srcbuf], rowbuf, sem).wait()

        def sg(t, _):
            rbase = t * 16
            sl = pl.ds(rbase, 16)
            ridx = _iota16() + rbase
            lanes = (dstbuf[sl] & 31) * 4
            wv4 = [estash[4 + h, sl] for h in range(4)]
            for h in range(4):
                plsc.store_scatter(denbuf, [ridx, lanes + h], wv4[h])
            for j in range(16):
                jj = rbase + j
                for h in range(4):
                    w = wv4[h][j]
                    for q in range(2):
                        sl2 = pl.ds(h * 32 + q * 16, 16)
                        rowbuf[jj, sl2] = rowbuf[jj, sl2] * w
            adid[sl] = adid[sl] + DEN0
            return 0

        lax.fori_loop(0, CH // 16, sg, 0)
        pltpu.sync_copy(rowbuf, acc_sh.at[dstbuf], add=True)
        pltpu.sync_copy(denbuf, acc_sh.at[adid], add=True)

        def zg(t, _):
            sl = pl.ds(t * 16, 16)
            ridx = _iota16() + t * 16
            lanes = (dstbuf[sl] & 31) * 4
            for h in range(4):
                plsc.store_scatter(denbuf, [ridx, lanes + h], zero16)
            return 0

        lax.fori_loop(0, CH // 16, zg, 0)
        return 0

    lax.fori_loop(0, EPT // CH, chunk, 0)
    plsc.subcore_barrier()
    for q in range(10):
        pltpu.sync_copy(acc_sh.at[pl.ds(r0 + q * CH, CH)], rowbuf)
        pltpu.sync_copy(rowbuf, out.at[c, pl.ds(r0 + q * CH, CH)])
    pltpu.sync_copy(acc_sh.at[pl.ds(d0, 32)], rowbuf.at[pl.ds(0, 32)])
    pltpu.sync_copy(rowbuf.at[pl.ds(0, 32)], out.at[c, pl.ds(d0, 32)])


@functools.cache
def _edge3_kernel():
    return pl.kernel(
        _edge3_body,
        out_type=jax.ShapeDtypeStruct((2, NPAD, 128), _f32),
        mesh=plsc.VectorSubcoreMesh(core_axis_name="c", subcore_axis_name="s"),
        compiler_params=pltpu.CompilerParams(needs_layout_passes=False),
        scratch_types=[
            pltpu.VMEM_SHARED((NPAD, 128), _f32),
            pltpu.VMEM((N,), _f32),
            pltpu.VMEM((N,), _f32),
            pltpu.VMEM((8, 128), _f32),
            pltpu.VMEM((CH,), jnp.int32),
            pltpu.VMEM((CH,), jnp.int32),
            pltpu.VMEM((CH, 128), _f32),
            pltpu.VMEM((1, CH), _f32),
            pltpu.SemaphoreType.DMA,
        ],
    )


def _edge3_body(hrows, srcf, dstf, ast, adt, mp, out, acc_sh, as_tab, ad_tab,
                m_vec, srcbuf, dstbuf, rowbuf, wtmp, sem):
    c = lax.axis_index("c")
    s = lax.axis_index("s")
    r0 = s * RPT

    _zero_rows(rowbuf, CH)
    for q in range(10):
        pltpu.sync_copy(rowbuf, acc_sh.at[pl.ds(r0 + q * CH, CH)])
    pltpu.sync_copy(ast, as_tab)
    pltpu.sync_copy(adt, ad_tab)
    pltpu.sync_copy(mp.at[0], m_vec)
    plsc.subcore_barrier()

    oh0 = (_iota16() == 0).astype(_f32)
    wid = s * 2 + c

    def chunk(k, _):
        base = wid * EPT3 + k * CH
        pltpu.sync_copy(srcf.at[pl.ds(base, CH)], srcbuf)
        pltpu.sync_copy(dstf.at[pl.ds(base, CH)], dstbuf)

        def wgroup(t, _):
            sl = pl.ds(t * 16, 16)
            asv = plsc.load_gather(as_tab, [srcbuf[sl]])
            adv = plsc.load_gather(ad_tab, [dstbuf[sl]])
            e = _leaky(asv + adv) - m_vec[0, pl.ds(0, 16)]
            wtmp[0, sl] = jnp.exp(e)
            return 0

        lax.fori_loop(0, CH // 16, wgroup, 0)
        pltpu.async_copy(hrows.at[srcbuf], rowbuf, sem).wait()

        def sgroup(t, _):
            rbase = t * 16
            wvec = wtmp[0, pl.ds(rbase, 16)]
            for j in range(16):
                jj = rbase + j
                w = wvec[j]
                for q in range(2):
                    sl2 = pl.ds(q * 16, 16)
                    rowbuf[jj, sl2] = rowbuf[jj, sl2] * w
                rowbuf[jj, pl.ds(32, 16)] = w * oh0
            return 0

        lax.fori_loop(0, CH // 16, sgroup, 0)
        pltpu.sync_copy(rowbuf, acc_sh.at[dstbuf], add=True)
        return 0

    lax.fori_loop(0, EPT3 // CH, chunk, 0)
    plsc.subcore_barrier()
    for q in range(10):
        pltpu.sync_copy(acc_sh.at[pl.ds(r0 + q * CH, CH)], rowbuf)
        pltpu.sync_copy(rowbuf, out.at[c, pl.ds(r0 + q * CH, CH)])


def _fold(W, a_s, a_d, heads):
    din = W.shape[0]
    wr = W.reshape(din, heads, HID)
    ws = jnp.einsum("dhc,hc->dh", wr, a_s)
    wd = jnp.einsum("dhc,hc->dh", wr, a_d)
    return ws, wd


def _den_view(acc):
    d = acc[:, DEN0:DEN0 + 320, :]
    return d.reshape(2, NPAD, 4)[:, :N, :]


def _pack_alpha(a):
    return jnp.pad(a, ((0, 0), (0, NPAD - N), (0, 0))).reshape(2, 320, 128)


def kernel(x, edge_index, batch, W1, a_src1, a_dst1, b1, W2, a_src2, a_dst2,
           b2, W3, a_src3, a_dst3, b3, Wl, bl):
    npad_e = EPAD - E
    src = jnp.concatenate([edge_index[0],
                           jnp.zeros((npad_e,), jnp.int32)])
    dst = jnp.concatenate([edge_index[1],
                           jnp.full((npad_e,), DUMMY_DST, jnp.int32)])

    ws1, wd1 = _fold(W1, a_src1, a_dst1, HEADS)
    ws2, wd2 = _fold(W2, a_src2, a_dst2, HEADS)
    ws3, wd3 = _fold(W3, a_src3, a_dst3, 1)
    w1h = jnp.stack([W1[:, 0:128], W1[:, 128:256]])
    w2h = jnp.stack([W2[:, 0:128], W2[:, 128:256]])
    wsd1 = jnp.stack([jnp.concatenate([ws1[:, 0:4], wd1[:, 0:4]], axis=1),
                      jnp.concatenate([ws1[:, 4:8], wd1[:, 4:8]], axis=1)])
    wsd2 = jnp.stack([jnp.concatenate([ws2[:, 0:4], wd2[:, 0:4]], axis=1),
                      jnp.concatenate([ws2[:, 4:8], wd2[:, 4:8]], axis=1)])
    w3a = jnp.concatenate([W3, ws3, jnp.zeros((W3.shape[0], 95), _f32)],
                          axis=1)

    h1, as1, ad1, m1 = _a1_call(x, w1h, wsd1)
    acc1 = _edge12_kernel()(h1, src, dst, _pack_alpha(as1),
                            _pack_alpha(ad1), m1)
    h2, as2, ad2, m2 = _a2_call(acc1, _den_view(acc1),
                                h1.reshape(2, N, 128), as1, ad1, m1,
                                b1.reshape(2, 128), w2h, wsd2)
    acc2 = _edge12_kernel()(h2, src, dst, _pack_alpha(as2),
                            _pack_alpha(ad2), m2)
    h3, ad3, m3 = _a3_call(acc2, _den_view(acc2),
                           h2.reshape(2, N, 128), as2, ad2, m2,
                           b2.reshape(2, 128), w3a, wd3)
    as3 = h3[:, 32:33]
    acc3 = _edge3_kernel()(h3, src, dst, as3.reshape(N), ad3.reshape(N), m3)
    out = _pool_call(acc3, h3, ad3, m3, b3.reshape(1, 32),
                     batch.reshape(N, 1), Wl, bl.reshape(1, OUT))
    return out


# speedup vs baseline: 28.0635x; 1.0441x over previous
import functools

import jax
import jax.numpy as jnp
from jax import lax
from jax.experimental import pallas as pl
from jax.experimental.pallas import tpu as pltpu
from jax.experimental.pallas import tpu_sc as plsc

N = 10000
E = 320000
D = 128
HID = 32
HEADS = 8
OUT = 10
G = 64

Bn = 1000
NB = N // Bn
CH = 64
EPAD = 321536
DUMMY_DST = 10200
EPT = EPAD // 16
EPT3 = EPAD // 32
NPAD = 10240
DEN0 = NPAD
NDEN = 512
NACC = NPAD + NDEN
RPT = NPAD // 16

_f32 = jnp.float32


def _leaky(x):
    return jnp.maximum(x, 0.2 * x)


def _elu(x):
    return jnp.where(x > 0, x, jnp.exp(jnp.minimum(x, 0.0)) - 1.0)


def _head_expand():
    lane = lax.broadcasted_iota(jnp.int32, (4, 128), 1) // 32
    head = lax.broadcasted_iota(jnp.int32, (4, 128), 0)
    return (lane == head).astype(_f32)


def _track_max(i, asb, adb, acc_ref, m_ref):
    smax = jnp.max(asb, axis=0, keepdims=True)
    dmax = jnp.max(adb, axis=0, keepdims=True)
    nh = asb.shape[1]

    @pl.when(i == 0)
    def _():
        acc_ref[0:1, 0:nh] = smax
        acc_ref[1:2, 0:nh] = dmax

    @pl.when(i > 0)
    def _():
        acc_ref[0:1, 0:nh] = jnp.maximum(acc_ref[0:1, 0:nh], smax)
        acc_ref[1:2, 0:nh] = jnp.maximum(acc_ref[1:2, 0:nh], dmax)

    @pl.when(i == NB - 1)
    def _():
        m = _leaky(acc_ref[0:1, 0:nh] + acc_ref[1:2, 0:nh])
        mpad = jnp.concatenate([m, jnp.zeros((1, 128 - nh), _f32)], axis=1)
        rowi = lax.broadcasted_iota(jnp.int32, (8, 128), 0)
        msplat = (rowi == 4).astype(_f32) * mpad
        for k in range(nh):
            msplat = msplat + (rowi == k).astype(_f32) * m[0:1, k:k + 1]
        m_ref[...] = msplat[None]


def _a1_body(x_ref, w_ref, wsd_ref, h_ref, ast_ref, adt_ref, m_ref, acc_ref):
    i = pl.program_id(1)
    xb = x_ref[...]
    hb = jnp.dot(xb, w_ref[0], preferred_element_type=_f32)
    h_ref[...] = hb
    sd = jnp.dot(xb, wsd_ref[0], preferred_element_type=_f32)
    ast_ref[0] = sd[:, 0:4]
    adt_ref[0] = sd[:, 4:8]
    _track_max(i, sd[:, 0:4], sd[:, 4:8], acc_ref, m_ref)


def _x_from_prev(acc_ref, den_ref, hprev_ref, asprev_ref, adprev_ref,
                 mprev_ref, b_ref):
    expand = _head_expand()
    parts = []
    for c in range(2):
        hb = hprev_ref[c]
        e = _leaky(asprev_ref[c] + adprev_ref[c])
        ws = jnp.exp(e - mprev_ref[c, 4:5, 0:4])
        den = den_ref[c] + ws
        repw = jnp.dot(ws, expand, preferred_element_type=_f32)
        repr_ = jnp.dot(1.0 / den, expand, preferred_element_type=_f32)
        xc = (acc_ref[c] + repw * hb) * repr_ + b_ref[c:c + 1, :]
        parts.append(_elu(xc))
    return jnp.concatenate(parts, axis=1)


def _a2_body(acc_ref, den_ref, hprev_ref, asprev_ref, adprev_ref, mprev_ref,
             b_ref, w_ref, wsd_ref, h_ref, ast_ref, adt_ref, m_ref, macc_ref):
    i = pl.program_id(1)
    x2 = _x_from_prev(acc_ref, den_ref, hprev_ref, asprev_ref, adprev_ref,
                      mprev_ref, b_ref)
    hb = jnp.dot(x2, w_ref[0], preferred_element_type=_f32)
    h_ref[...] = hb
    sd = jnp.dot(x2, wsd_ref[0], preferred_element_type=_f32)
    ast_ref[0] = sd[:, 0:4]
    adt_ref[0] = sd[:, 4:8]
    _track_max(i, sd[:, 0:4], sd[:, 4:8], macc_ref, m_ref)


def _track_max_one(i, asb, adb, acc_ref, m_ref):
    smax = jnp.max(asb, axis=0, keepdims=True)
    dmax = jnp.max(adb, axis=0, keepdims=True)

    @pl.when(i == 0)
    def _():
        acc_ref[0:1, 0:1] = smax
        acc_ref[1:2, 0:1] = dmax

    @pl.when(i > 0)
    def _():
        acc_ref[0:1, 0:1] = jnp.maximum(acc_ref[0:1, 0:1], smax)
        acc_ref[1:2, 0:1] = jnp.maximum(acc_ref[1:2, 0:1], dmax)

    @pl.when(i == NB - 1)
    def _():
        m = _leaky(acc_ref[0:1, 0:1] + acc_ref[1:2, 0:1])
        mpad = jnp.concatenate([m, jnp.zeros((1, 127), _f32)], axis=1)
        rowi = lax.broadcasted_iota(jnp.int32, (8, 128), 0)
        msplat = ((rowi == 4).astype(_f32) * mpad
                  + (rowi == 0).astype(_f32) * m[0:1, 0:1])
        m_ref[...] = msplat[None]


def _a3_body(acc_ref, den_ref, hprev_ref, asprev_ref, adprev_ref, mprev_ref,
             b_ref, w_ref, wd_ref, h_ref, adt_ref, m_ref, macc_ref):
    i = pl.program_id(0)
    x3 = _x_from_prev(acc_ref, den_ref, hprev_ref, asprev_ref, adprev_ref,
                      mprev_ref, b_ref)
    hb = jnp.dot(x3, w_ref[...], preferred_element_type=_f32)
    h_ref[...] = hb
    adb = jnp.dot(x3, wd_ref[...], preferred_element_type=_f32)
    adt_ref[...] = adb
    _track_max_one(i, hb[:, 32:33], adb, macc_ref, m_ref)


def _pool_body(acc_ref, hprev_ref, adprev_ref, mprev_ref, b3_ref, batch_ref,
               wl_ref, bl_ref, out_ref, sum_ref, cnt_ref):
    i = pl.program_id(0)
    hb = hprev_ref[...]
    e = _leaky(hb[:, 32:33] + adprev_ref[...])
    ws = jnp.exp(e - mprev_ref[0, 4:5, 0:1])
    den = acc_ref[0][:, 32:33] + acc_ref[1][:, 32:33] + ws
    num = acc_ref[0][:, 0:32] + acc_ref[1][:, 0:32] + ws * hb[:, 0:32]
    r = jnp.maximum(num / den + b3_ref[...], 0.0)

    bb = batch_ref[...]
    onehot = (lax.broadcasted_iota(jnp.int32, (Bn, G), 1)
              == bb).astype(_f32)
    dn = (((0,), (0,)), ((), ()))
    psum = lax.dot_general(onehot, r, dn, preferred_element_type=_f32)
    pcnt = lax.dot_general(onehot, jnp.ones((Bn, 32), _f32), dn,
                           preferred_element_type=_f32)

    @pl.when(i == 0)
    def _():
        sum_ref[...] = psum
        cnt_ref[...] = pcnt

    @pl.when(i > 0)
    def _():
        sum_ref[...] = sum_ref[...] + psum
        cnt_ref[...] = cnt_ref[...] + pcnt

    @pl.when(i == NB - 1)
    def _():
        pooled = sum_ref[...] / jnp.maximum(cnt_ref[...], 1.0)
        out_ref[...] = (jnp.dot(pooled, wl_ref[...],
                                preferred_element_type=_f32) + bl_ref[...])


def _a1_call(x, w, wsd):
    return pl.pallas_call(
        _a1_body,
        grid=(2, NB),
        in_specs=[
            pl.BlockSpec((Bn, 128), lambda j, i: (i, 0)),
            pl.BlockSpec((1, 128, 128), lambda j, i: (j, 0, 0)),
            pl.BlockSpec((1, 128, 8), lambda j, i: (j, 0, 0)),
        ],
        out_specs=[
            pl.BlockSpec((Bn, 128), lambda j, i: (j * NB + i, 0)),
            pl.BlockSpec((1, Bn, 4), lambda j, i: (j, i, 0)),
            pl.BlockSpec((1, Bn, 4), lambda j, i: (j, i, 0)),
            pl.BlockSpec((1, 8, 128), lambda j, i: (j, 0, 0)),
        ],
        out_shape=[
            jax.ShapeDtypeStruct((2 * N, 128), _f32),
            jax.ShapeDtypeStruct((2, N, 4), _f32),
            jax.ShapeDtypeStruct((2, N, 4), _f32),
            jax.ShapeDtypeStruct((2, 8, 128), _f32),
        ],
        scratch_shapes=[pltpu.VMEM((8, 128), _f32)],
    )(x, w, wsd)


def _a2_call(acc, den, hprev, asprev, adprev, mprev, b, w, wsd):
    return pl.pallas_call(
        _a2_body,
        grid=(2, NB),
        in_specs=[
            pl.BlockSpec((2, Bn, 128), lambda j, i: (0, i, 0)),
            pl.BlockSpec((2, Bn, 4), lambda j, i: (0, i, 0)),
            pl.BlockSpec((2, Bn, 128), lambda j, i: (0, i, 0)),
            pl.BlockSpec((2, Bn, 4), lambda j, i: (0, i, 0)),
            pl.BlockSpec((2, Bn, 4), lambda j, i: (0, i, 0)),
            pl.BlockSpec((2, 8, 128), lambda j, i: (0, 0, 0)),
            pl.BlockSpec((2, 128), lambda j, i: (0, 0)),
            pl.BlockSpec((1, 256, 128), lambda j, i: (j, 0, 0)),
            pl.BlockSpec((1, 256, 8), lambda j, i: (j, 0, 0)),
        ],
        out_specs=[
            pl.BlockSpec((Bn, 128), lambda j, i: (j * NB + i, 0)),
            pl.BlockSpec((1, Bn, 4), lambda j, i: (j, i, 0)),
            pl.BlockSpec((1, Bn, 4), lambda j, i: (j, i, 0)),
            pl.BlockSpec((1, 8, 128), lambda j, i: (j, 0, 0)),
        ],
        out_shape=[
            jax.ShapeDtypeStruct((2 * N, 128), _f32),
            jax.ShapeDtypeStruct((2, N, 4), _f32),
            jax.ShapeDtypeStruct((2, N, 4), _f32),
            jax.ShapeDtypeStruct((2, 8, 128), _f32),
        ],
        scratch_shapes=[pltpu.VMEM((8, 128), _f32)],
    )(acc, den, hprev, asprev, adprev, mprev, b, w, wsd)


def _a3_call(acc, den, hprev, asprev, adprev, mprev, b, w, wd):
    return pl.pallas_call(
        _a3_body,
        grid=(NB,),
        in_specs=[
            pl.BlockSpec((2, Bn, 128), lambda i: (0, i, 0)),
            pl.BlockSpec((2, Bn, 4), lambda i: (0, i, 0)),
            pl.BlockSpec((2, Bn, 128), lambda i: (0, i, 0)),
            pl.BlockSpec((2, Bn, 4), lambda i: (0, i, 0)),
            pl.BlockSpec((2, Bn, 4), lambda i: (0, i, 0)),
            pl.BlockSpec((2, 8, 128), lambda i: (0, 0, 0)),
            pl.BlockSpec((2, 128), lambda i: (0, 0)),
            pl.BlockSpec((256, 128), lambda i: (0, 0)),
            pl.BlockSpec((256, 1), lambda i: (0, 0)),
        ],
        out_specs=[
            pl.BlockSpec((Bn, 128), lambda i: (i, 0)),
            pl.BlockSpec((Bn, 1), lambda i: (i, 0)),
            pl.BlockSpec((1, 8, 128), lambda i: (0, 0, 0)),
        ],
        out_shape=[
            jax.ShapeDtypeStruct((N, 128), _f32),
            jax.ShapeDtypeStruct((N, 1), _f32),
            jax.ShapeDtypeStruct((1, 8, 128), _f32),
        ],
        scratch_shapes=[pltpu.VMEM((8, 128), _f32)],
    )(acc, den, hprev, asprev, adprev, mprev, b, w, wd)


def _pool_call(acc3, h3, adt3, m3, b3, batch, wl, bl):
    return pl.pallas_call(
        _pool_body,
        grid=(NB,),
        in_specs=[
            pl.BlockSpec((2, Bn, 128), lambda i: (0, i, 0)),
            pl.BlockSpec((Bn, 128), lambda i: (i, 0)),
            pl.BlockSpec((Bn, 1), lambda i: (i, 0)),
            pl.BlockSpec((1, 8, 128), lambda i: (0, 0, 0)),
            pl.BlockSpec((1, 32), lambda i: (0, 0)),
            pl.BlockSpec((Bn, 1), lambda i: (i, 0)),
            pl.BlockSpec((32, OUT), lambda i: (0, 0)),
            pl.BlockSpec((1, OUT), lambda i: (0, 0)),
        ],
        out_specs=pl.BlockSpec((G, OUT), lambda i: (0, 0)),
        out_shape=jax.ShapeDtypeStruct((G, OUT), _f32),
        scratch_shapes=[pltpu.VMEM((G, 32), _f32), pltpu.VMEM((G, 32), _f32)],
    )(acc3, h3, adt3, m3, b3, batch, wl, bl)


def _iota16():
    return lax.broadcasted_iota(jnp.int32, (16,), 0)


def _zero_rows(buf, nrows):
    z = jnp.zeros((16,), _f32)

    def body(i, _):
        for q in range(8):
            buf[i, pl.ds(q * 16, 16)] = z
        return 0

    lax.fori_loop(0, nrows, body, 0)


@functools.cache
def _edge12_kernel():
    return pl.kernel(
        _edge12_body,
        out_type=jax.ShapeDtypeStruct((2, NACC, 128), _f32),
        mesh=plsc.VectorSubcoreMesh(core_axis_name="c", subcore_axis_name="s"),
        compiler_params=pltpu.CompilerParams(needs_layout_passes=False),
        scratch_types=[
            pltpu.VMEM_SHARED((NACC, 128), _f32),
            pltpu.VMEM_SHARED((320, 128), _f32),
            pltpu.VMEM_SHARED((320, 128), _f32),
            pltpu.VMEM((8, 128), _f32),
            pltpu.VMEM((CH,), jnp.int32),
            pltpu.VMEM((CH,), jnp.int32),
            pltpu.VMEM((CH,), jnp.int32),
            pltpu.VMEM((CH,), jnp.int32),
            pltpu.VMEM((CH,), jnp.int32),
            pltpu.VMEM((CH, 128), _f32),
            pltpu.VMEM((CH, 128), _f32),
            pltpu.VMEM((CH, 128), _f32),
            pltpu.VMEM((CH, 128), _f32),
            pltpu.SemaphoreType.DMA,
            pltpu.SemaphoreType.DMA,
        ],
    )


def _edge12_body(hrows, srcf, dstf, aspk, adpk, mp, out, acc_sh, as_sh, ad_sh,
                 m_vec, srcbuf, dstbuf, asid, adid, hidx, asg, adg, rowbuf,
                 denbuf, gsem, ssem):
    c = lax.axis_index("c")
    s = lax.axis_index("s")
    r0 = s * RPT
    d0 = DEN0 + s * (NDEN // 16)

    @pl.when(s == 0)
    def _():
        pltpu.sync_copy(aspk.at[c], as_sh)
        pltpu.sync_copy(adpk.at[c], ad_sh)

    _zero_rows(rowbuf, CH)
    _zero_rows(denbuf, CH)
    for q in range(10):
        pltpu.sync_copy(rowbuf, acc_sh.at[pl.ds(r0 + q * CH, CH)])
    pltpu.sync_copy(rowbuf.at[pl.ds(0, 32)], acc_sh.at[pl.ds(d0, 32)])
    pltpu.sync_copy(mp.at[c], m_vec)
    plsc.subcore_barrier()

    c_n = c * N
    zero16 = jnp.zeros((16,), _f32)

    def chunk(k, _):
        base = s * EPT + k * CH
        pltpu.sync_copy(srcf.at[pl.ds(base, CH)], srcbuf)
        pltpu.sync_copy(dstf.at[pl.ds(base, CH)], dstbuf)

        def prep(t, _):
            sl = pl.ds(t * 16, 16)
            sv = srcbuf[sl]
            asid[sl] = lax.shift_right_logical(sv, 5)
            adid[sl] = lax.shift_right_logical(dstbuf[sl], 5)
            hidx[sl] = sv + c_n
            return 0

        lax.fori_loop(0, CH // 16, prep, 0)

        pltpu.async_copy(as_sh.at[asid], asg, gsem).wait()
        pltpu.async_copy(ad_sh.at[adid], adg, gsem).wait()
        pltpu.async_copy(hrows.at[hidx], rowbuf, gsem).wait()

        def sg(t, _):
            rbase = t * 16
            sl = pl.ds(rbase, 16)
            ridx = _iota16() + rbase
            lsrc = (srcbuf[sl] & 31) * 4
            lanes = (dstbuf[sl] & 31) * 4
            wv4 = []
            for h in range(4):
                asv = plsc.load_gather(asg, [ridx, lsrc + h])
                adv = plsc.load_gather(adg, [ridx, lanes + h])
                e = _leaky(asv + adv) - m_vec[h, pl.ds(0, 16)]
                wv4.append(jnp.exp(e))
            for h in range(4):
                plsc.store_scatter(denbuf, [ridx, lanes + h], wv4[h])
            for j in range(16):
                jj = rbase + j
                for h in range(4):
                    w = wv4[h][j]
                    for q in range(2):
                        sl2 = pl.ds(h * 32 + q * 16, 16)
                        rowbuf[jj, sl2] = rowbuf[jj, sl2] * w
            adid[sl] = adid[sl] + DEN0
            return 0

        lax.fori_loop(0, CH // 16, sg, 0)
        pltpu.sync_copy(rowbuf, acc_sh.at[dstbuf], add=True)
        pltpu.sync_copy(denbuf, acc_sh.at[adid], add=True)

        def zg(t, _):
            sl = pl.ds(t * 16, 16)
            ridx = _iota16() + t * 16
            lanes = (dstbuf[sl] & 31) * 4
            for h in range(4):
                plsc.store_scatter(denbuf, [ridx, lanes + h], zero16)
            return 0

        lax.fori_loop(0, CH // 16, zg, 0)
        return 0

    lax.fori_loop(0, EPT // CH, chunk, 0)
    plsc.subcore_barrier()
    for q in range(10):
        pltpu.sync_copy(acc_sh.at[pl.ds(r0 + q * CH, CH)], rowbuf)
        pltpu.sync_copy(rowbuf, out.at[c, pl.ds(r0 + q * CH, CH)])
    pltpu.sync_copy(acc_sh.at[pl.ds(d0, 32)], rowbuf.at[pl.ds(0, 32)])
    pltpu.sync_copy(rowbuf.at[pl.ds(0, 32)], out.at[c, pl.ds(d0, 32)])


@functools.cache
def _edge3_kernel():
    return pl.kernel(
        _edge3_body,
        out_type=jax.ShapeDtypeStruct((2, NPAD, 128), _f32),
        mesh=plsc.VectorSubcoreMesh(core_axis_name="c", subcore_axis_name="s"),
        compiler_params=pltpu.CompilerParams(needs_layout_passes=False),
        scratch_types=[
            pltpu.VMEM_SHARED((NPAD, 128), _f32),
            pltpu.VMEM((N,), _f32),
            pltpu.VMEM((N,), _f32),
            pltpu.VMEM((8, 128), _f32),
            pltpu.VMEM((CH,), jnp.int32),
            pltpu.VMEM((CH,), jnp.int32),
            pltpu.VMEM((CH, 128), _f32),
            pltpu.VMEM((1, CH), _f32),
            pltpu.SemaphoreType.DMA,
        ],
    )


def _edge3_body(hrows, srcf, dstf, ast, adt, mp, out, acc_sh, as_tab, ad_tab,
                m_vec, srcbuf, dstbuf, rowbuf, wtmp, sem):
    c = lax.axis_index("c")
    s = lax.axis_index("s")
    r0 = s * RPT

    _zero_rows(rowbuf, CH)
    for q in range(10):
        pltpu.sync_copy(rowbuf, acc_sh.at[pl.ds(r0 + q * CH, CH)])
    pltpu.sync_copy(ast, as_tab)
    pltpu.sync_copy(adt, ad_tab)
    pltpu.sync_copy(mp.at[0], m_vec)
    plsc.subcore_barrier()

    oh0 = (_iota16() == 0).astype(_f32)
    wid = s * 2 + c

    def chunk(k, _):
        base = wid * EPT3 + k * CH
        pltpu.sync_copy(srcf.at[pl.ds(base, CH)], srcbuf)
        pltpu.sync_copy(dstf.at[pl.ds(base, CH)], dstbuf)

        def wgroup(t, _):
            sl = pl.ds(t * 16, 16)
            asv = plsc.load_gather(as_tab, [srcbuf[sl]])
            adv = plsc.load_gather(ad_tab, [dstbuf[sl]])
            e = _leaky(asv + adv) - m_vec[0, pl.ds(0, 16)]
            wtmp[0, sl] = jnp.exp(e)
            return 0

        lax.fori_loop(0, CH // 16, wgroup, 0)
        pltpu.async_copy(hrows.at[srcbuf], rowbuf, sem).wait()

        def sgroup(t, _):
            rbase = t * 16
            wvec = wtmp[0, pl.ds(rbase, 16)]
            for j in range(16):
                jj = rbase + j
                w = wvec[j]
                for q in range(2):
                    sl2 = pl.ds(q * 16, 16)
                    rowbuf[jj, sl2] = rowbuf[jj, sl2] * w
                rowbuf[jj, pl.ds(32, 16)] = w * oh0
            return 0

        lax.fori_loop(0, CH // 16, sgroup, 0)
        pltpu.sync_copy(rowbuf, acc_sh.at[dstbuf], add=True)
        return 0

    lax.fori_loop(0, EPT3 // CH, chunk, 0)
    plsc.subcore_barrier()
    for q in range(10):
        pltpu.sync_copy(acc_sh.at[pl.ds(r0 + q * CH, CH)], rowbuf)
        pltpu.sync_copy(rowbuf, out.at[c, pl.ds(r0 + q * CH, CH)])


def _fold(W, a_s, a_d, heads):
    din = W.shape[0]
    wr = W.reshape(din, heads, HID)
    ws = jnp.einsum("dhc,hc->dh", wr, a_s)
    wd = jnp.einsum("dhc,hc->dh", wr, a_d)
    return ws, wd


def _den_view(acc):
    d = acc[:, DEN0:DEN0 + 320, :]
    return d.reshape(2, NPAD, 4)[:, :N, :]


def _pack_alpha(a):
    return jnp.pad(a, ((0, 0), (0, NPAD - N), (0, 0))).reshape(2, 320, 128)


def kernel(x, edge_index, batch, W1, a_src1, a_dst1, b1, W2, a_src2, a_dst2,
           b2, W3, a_src3, a_dst3, b3, Wl, bl):
    npad_e = EPAD - E
    src = jnp.concatenate([edge_index[0],
                           jnp.zeros((npad_e,), jnp.int32)])
    dst = jnp.concatenate([edge_index[1],
                           jnp.full((npad_e,), DUMMY_DST, jnp.int32)])

    ws1, wd1 = _fold(W1, a_src1, a_dst1, HEADS)
    ws2, wd2 = _fold(W2, a_src2, a_dst2, HEADS)
    ws3, wd3 = _fold(W3, a_src3, a_dst3, 1)
    w1h = jnp.stack([W1[:, 0:128], W1[:, 128:256]])
    w2h = jnp.stack([W2[:, 0:128], W2[:, 128:256]])
    wsd1 = jnp.stack([jnp.concatenate([ws1[:, 0:4], wd1[:, 0:4]], axis=1),
                      jnp.concatenate([ws1[:, 4:8], wd1[:, 4:8]], axis=1)])
    wsd2 = jnp.stack([jnp.concatenate([ws2[:, 0:4], wd2[:, 0:4]], axis=1),
                      jnp.concatenate([ws2[:, 4:8], wd2[:, 4:8]], axis=1)])
    w3a = jnp.concatenate([W3, ws3, jnp.zeros((W3.shape[0], 95), _f32)],
                          axis=1)

    h1, as1, ad1, m1 = _a1_call(x, w1h, wsd1)
    acc1 = _edge12_kernel()(h1, src, dst, _pack_alpha(as1),
                            _pack_alpha(ad1), m1)
    h2, as2, ad2, m2 = _a2_call(acc1, _den_view(acc1),
                                h1.reshape(2, N, 128), as1, ad1, m1,
                                b1.reshape(2, 128), w2h, wsd2)
    acc2 = _edge12_kernel()(h2, src, dst, _pack_alpha(as2),
                            _pack_alpha(ad2), m2)
    h3, ad3, m3 = _a3_call(acc2, _den_view(acc2),
                           h2.reshape(2, N, 128), as2, ad2, m2,
                           b2.reshape(2, 128), w3a, wd3)
    as3 = h3[:, 32:33]
    acc3 = _edge3_kernel()(h3, src, dst, as3.reshape(N), ad3.reshape(N), m3)
    out = _pool_call(acc3, h3, ad3, m3, b3.reshape(1, 32),
                     batch.reshape(N, 1), Wl, bl.reshape(1, OUT))
    return out


# speedup vs baseline: 34.5555x; 1.2313x over previous
import functools

import jax
import jax.numpy as jnp
from jax import lax
from jax.experimental import pallas as pl
from jax.experimental.pallas import tpu as pltpu
from jax.experimental.pallas import tpu_sc as plsc

N = 10000
E = 320000
D = 128
HID = 32
HEADS = 8
OUT = 10
G = 64

Bn = 1000
NB = N // Bn
CH = 64
EPAD = 321536
DUMMY_DST = 10200
EPT = EPAD // 16
EPT3 = EPAD // 32
NPAD = 10240
DEN0 = NPAD
NDEN = 512
NACC = NPAD + NDEN
RPT = NPAD // 16

_f32 = jnp.float32


def _leaky(x):
    return jnp.maximum(x, 0.2 * x)


def _elu(x):
    return jnp.where(x > 0, x, jnp.exp(jnp.minimum(x, 0.0)) - 1.0)


def _head_expand():
    lane = lax.broadcasted_iota(jnp.int32, (4, 128), 1) // 32
    head = lax.broadcasted_iota(jnp.int32, (4, 128), 0)
    return (lane == head).astype(_f32)


def _track_max(i, asb, adb, acc_ref, m_ref):
    smax = jnp.max(asb, axis=0, keepdims=True)
    dmax = jnp.max(adb, axis=0, keepdims=True)
    nh = asb.shape[1]

    @pl.when(i == 0)
    def _():
        acc_ref[0:1, 0:nh] = smax
        acc_ref[1:2, 0:nh] = dmax

    @pl.when(i > 0)
    def _():
        acc_ref[0:1, 0:nh] = jnp.maximum(acc_ref[0:1, 0:nh], smax)
        acc_ref[1:2, 0:nh] = jnp.maximum(acc_ref[1:2, 0:nh], dmax)

    @pl.when(i == NB - 1)
    def _():
        m = _leaky(acc_ref[0:1, 0:nh] + acc_ref[1:2, 0:nh])
        mpad = jnp.concatenate([m, jnp.zeros((1, 128 - nh), _f32)], axis=1)
        rowi = lax.broadcasted_iota(jnp.int32, (8, 128), 0)
        msplat = (rowi == 4).astype(_f32) * mpad
        for k in range(nh):
            msplat = msplat + (rowi == k).astype(_f32) * m[0:1, k:k + 1]
        m_ref[...] = msplat[None]


def _a1_body(x_ref, w_ref, wsd_ref, h_ref, ast_ref, adt_ref, m_ref, acc_ref):
    i = pl.program_id(1)
    xb = x_ref[...]
    hb = jnp.dot(xb, w_ref[0], preferred_element_type=_f32)
    h_ref[...] = hb
    sd = jnp.dot(xb, wsd_ref[0], preferred_element_type=_f32)
    ast_ref[0] = sd[:, 0:4]
    adt_ref[0] = sd[:, 4:8]
    _track_max(i, sd[:, 0:4], sd[:, 4:8], acc_ref, m_ref)


def _x_from_prev(acc_ref, den_ref, hprev_ref, asprev_ref, adprev_ref,
                 mprev_ref, b_ref):
    expand = _head_expand()
    parts = []
    for c in range(2):
        hb = hprev_ref[c]
        e = _leaky(asprev_ref[c] + adprev_ref[c])
        ws = jnp.exp(e - mprev_ref[c, 4:5, 0:4])
        den = den_ref[c] + ws
        repw = jnp.dot(ws, expand, preferred_element_type=_f32)
        repr_ = jnp.dot(1.0 / den, expand, preferred_element_type=_f32)
        xc = (acc_ref[c] + repw * hb) * repr_ + b_ref[c:c + 1, :]
        parts.append(_elu(xc))
    return jnp.concatenate(parts, axis=1)


def _a2_body(acc_ref, den_ref, hprev_ref, asprev_ref, adprev_ref, mprev_ref,
             b_ref, w_ref, wsd_ref, h_ref, ast_ref, adt_ref, m_ref, macc_ref):
    i = pl.program_id(1)
    x2 = _x_from_prev(acc_ref, den_ref, hprev_ref, asprev_ref, adprev_ref,
                      mprev_ref, b_ref)
    hb = jnp.dot(x2, w_ref[0], preferred_element_type=_f32)
    h_ref[...] = hb
    sd = jnp.dot(x2, wsd_ref[0], preferred_element_type=_f32)
    ast_ref[0] = sd[:, 0:4]
    adt_ref[0] = sd[:, 4:8]
    _track_max(i, sd[:, 0:4], sd[:, 4:8], macc_ref, m_ref)


def _track_max_one(i, asb, adb, acc_ref, m_ref):
    smax = jnp.max(asb, axis=0, keepdims=True)
    dmax = jnp.max(adb, axis=0, keepdims=True)

    @pl.when(i == 0)
    def _():
        acc_ref[0:1, 0:1] = smax
        acc_ref[1:2, 0:1] = dmax

    @pl.when(i > 0)
    def _():
        acc_ref[0:1, 0:1] = jnp.maximum(acc_ref[0:1, 0:1], smax)
        acc_ref[1:2, 0:1] = jnp.maximum(acc_ref[1:2, 0:1], dmax)

    @pl.when(i == NB - 1)
    def _():
        m = _leaky(acc_ref[0:1, 0:1] + acc_ref[1:2, 0:1])
        mpad = jnp.concatenate([m, jnp.zeros((1, 127), _f32)], axis=1)
        rowi = lax.broadcasted_iota(jnp.int32, (8, 128), 0)
        msplat = ((rowi == 4).astype(_f32) * mpad
                  + (rowi == 0).astype(_f32) * m[0:1, 0:1])
        m_ref[...] = msplat[None]


def _a3_body(acc_ref, den_ref, hprev_ref, asprev_ref, adprev_ref, mprev_ref,
             b_ref, w_ref, wd_ref, h_ref, adt_ref, m_ref, macc_ref):
    i = pl.program_id(0)
    x3 = _x_from_prev(acc_ref, den_ref, hprev_ref, asprev_ref, adprev_ref,
                      mprev_ref, b_ref)
    hb = jnp.dot(x3, w_ref[...], preferred_element_type=_f32)
    h_ref[...] = hb
    adb = jnp.dot(x3, wd_ref[...], preferred_element_type=_f32)
    adt_ref[...] = adb
    _track_max_one(i, hb[:, 32:33], adb, macc_ref, m_ref)


def _pool_body(acc_ref, hprev_ref, adprev_ref, mprev_ref, b3_ref, batch_ref,
               wl_ref, bl_ref, out_ref, sum_ref, cnt_ref):
    i = pl.program_id(0)
    hb = hprev_ref[...]
    e = _leaky(hb[:, 32:33] + adprev_ref[...])
    ws = jnp.exp(e - mprev_ref[0, 4:5, 0:1])
    den = acc_ref[0][:, 32:33] + acc_ref[1][:, 32:33] + ws
    num = acc_ref[0][:, 0:32] + acc_ref[1][:, 0:32] + ws * hb[:, 0:32]
    r = jnp.maximum(num / den + b3_ref[...], 0.0)

    bb = batch_ref[...]
    onehot = (lax.broadcasted_iota(jnp.int32, (Bn, G), 1)
              == bb).astype(_f32)
    dn = (((0,), (0,)), ((), ()))
    psum = lax.dot_general(onehot, r, dn, preferred_element_type=_f32)
    pcnt = lax.dot_general(onehot, jnp.ones((Bn, 32), _f32), dn,
                           preferred_element_type=_f32)

    @pl.when(i == 0)
    def _():
        sum_ref[...] = psum
        cnt_ref[...] = pcnt

    @pl.when(i > 0)
    def _():
        sum_ref[...] = sum_ref[...] + psum
        cnt_ref[...] = cnt_ref[...] + pcnt

    @pl.when(i == NB - 1)
    def _():
        pooled = sum_ref[...] / jnp.maximum(cnt_ref[...], 1.0)
        out_ref[...] = (jnp.dot(pooled, wl_ref[...],
                                preferred_element_type=_f32) + bl_ref[...])


def _a1_call(x, w, wsd):
    return pl.pallas_call(
        _a1_body,
        grid=(2, NB),
        in_specs=[
            pl.BlockSpec((Bn, 128), lambda j, i: (i, 0)),
            pl.BlockSpec((1, 128, 128), lambda j, i: (j, 0, 0)),
            pl.BlockSpec((1, 128, 8), lambda j, i: (j, 0, 0)),
        ],
        out_specs=[
            pl.BlockSpec((Bn, 128), lambda j, i: (j * NB + i, 0)),
            pl.BlockSpec((1, Bn, 4), lambda j, i: (j, i, 0)),
            pl.BlockSpec((1, Bn, 4), lambda j, i: (j, i, 0)),
            pl.BlockSpec((1, 8, 128), lambda j, i: (j, 0, 0)),
        ],
        out_shape=[
            jax.ShapeDtypeStruct((2 * N, 128), _f32),
            jax.ShapeDtypeStruct((2, N, 4), _f32),
            jax.ShapeDtypeStruct((2, N, 4), _f32),
            jax.ShapeDtypeStruct((2, 8, 128), _f32),
        ],
        scratch_shapes=[pltpu.VMEM((8, 128), _f32)],
    )(x, w, wsd)


def _a2_call(acc, den, hprev, asprev, adprev, mprev, b, w, wsd):
    return pl.pallas_call(
        _a2_body,
        grid=(2, NB),
        in_specs=[
            pl.BlockSpec((2, Bn, 128), lambda j, i: (0, i, 0)),
            pl.BlockSpec((2, Bn, 4), lambda j, i: (0, i, 0)),
            pl.BlockSpec((2, Bn, 128), lambda j, i: (0, i, 0)),
            pl.BlockSpec((2, Bn, 4), lambda j, i: (0, i, 0)),
            pl.BlockSpec((2, Bn, 4), lambda j, i: (0, i, 0)),
            pl.BlockSpec((2, 8, 128), lambda j, i: (0, 0, 0)),
            pl.BlockSpec((2, 128), lambda j, i: (0, 0)),
            pl.BlockSpec((1, 256, 128), lambda j, i: (j, 0, 0)),
            pl.BlockSpec((1, 256, 8), lambda j, i: (j, 0, 0)),
        ],
        out_specs=[
            pl.BlockSpec((Bn, 128), lambda j, i: (j * NB + i, 0)),
            pl.BlockSpec((1, Bn, 4), lambda j, i: (j, i, 0)),
            pl.BlockSpec((1, Bn, 4), lambda j, i: (j, i, 0)),
            pl.BlockSpec((1, 8, 128), lambda j, i: (j, 0, 0)),
        ],
        out_shape=[
            jax.ShapeDtypeStruct((2 * N, 128), _f32),
            jax.ShapeDtypeStruct((2, N, 4), _f32),
            jax.ShapeDtypeStruct((2, N, 4), _f32),
            jax.ShapeDtypeStruct((2, 8, 128), _f32),
        ],
        scratch_shapes=[pltpu.VMEM((8, 128), _f32)],
    )(acc, den, hprev, asprev, adprev, mprev, b, w, wsd)


def _a3_call(acc, den, hprev, asprev, adprev, mprev, b, w, wd):
    return pl.pallas_call(
        _a3_body,
        grid=(NB,),
        in_specs=[
            pl.BlockSpec((2, Bn, 128), lambda i: (0, i, 0)),
            pl.BlockSpec((2, Bn, 4), lambda i: (0, i, 0)),
            pl.BlockSpec((2, Bn, 128), lambda i: (0, i, 0)),
            pl.BlockSpec((2, Bn, 4), lambda i: (0, i, 0)),
            pl.BlockSpec((2, Bn, 4), lambda i: (0, i, 0)),
            pl.BlockSpec((2, 8, 128), lambda i: (0, 0, 0)),
            pl.BlockSpec((2, 128), lambda i: (0, 0)),
            pl.BlockSpec((256, 128), lambda i: (0, 0)),
            pl.BlockSpec((256, 1), lambda i: (0, 0)),
        ],
        out_specs=[
            pl.BlockSpec((Bn, 128), lambda i: (i, 0)),
            pl.BlockSpec((Bn, 1), lambda i: (i, 0)),
            pl.BlockSpec((1, 8, 128), lambda i: (0, 0, 0)),
        ],
        out_shape=[
            jax.ShapeDtypeStruct((N, 128), _f32),
            jax.ShapeDtypeStruct((N, 1), _f32),
            jax.ShapeDtypeStruct((1, 8, 128), _f32),
        ],
        scratch_shapes=[pltpu.VMEM((8, 128), _f32)],
    )(acc, den, hprev, asprev, adprev, mprev, b, w, wd)


def _pool_call(acc3, h3, adt3, m3, b3, batch, wl, bl):
    return pl.pallas_call(
        _pool_body,
        grid=(NB,),
        in_specs=[
            pl.BlockSpec((2, Bn, 128), lambda i: (0, i, 0)),
            pl.BlockSpec((Bn, 128), lambda i: (i, 0)),
            pl.BlockSpec((Bn, 1), lambda i: (i, 0)),
            pl.BlockSpec((1, 8, 128), lambda i: (0, 0, 0)),
            pl.BlockSpec((1, 32), lambda i: (0, 0)),
            pl.BlockSpec((Bn, 1), lambda i: (i, 0)),
            pl.BlockSpec((32, OUT), lambda i: (0, 0)),
            pl.BlockSpec((1, OUT), lambda i: (0, 0)),
        ],
        out_specs=pl.BlockSpec((G, OUT), lambda i: (0, 0)),
        out_shape=jax.ShapeDtypeStruct((G, OUT), _f32),
        scratch_shapes=[pltpu.VMEM((G, 32), _f32), pltpu.VMEM((G, 32), _f32)],
    )(acc3, h3, adt3, m3, b3, batch, wl, bl)


def _iota16():
    return lax.broadcasted_iota(jnp.int32, (16,), 0)


def _zero_rows(buf, nrows):
    z = jnp.zeros((16,), _f32)

    def body(i, _):
        for q in range(8):
            buf[i, pl.ds(q * 16, 16)] = z
        return 0

    lax.fori_loop(0, nrows, body, 0)


@functools.cache
def _edge12_kernel():
    return pl.kernel(
        _edge12_body,
        out_type=jax.ShapeDtypeStruct((2, NACC, 128), _f32),
        mesh=plsc.VectorSubcoreMesh(core_axis_name="c", subcore_axis_name="s"),
        compiler_params=pltpu.CompilerParams(needs_layout_passes=False),
        scratch_types=[
            pltpu.VMEM_SHARED((NACC, 128), _f32),
            pltpu.VMEM_SHARED((320, 128), _f32),
            pltpu.VMEM_SHARED((320, 128), _f32),
            pltpu.VMEM((8, 128), _f32),
            pltpu.VMEM((CH,), jnp.int32),
            pltpu.VMEM((CH,), jnp.int32),
            pltpu.VMEM((CH,), jnp.int32),
            pltpu.VMEM((CH,), jnp.int32),
            pltpu.VMEM((CH,), jnp.int32),
            pltpu.VMEM((CH, 128), _f32),
            pltpu.VMEM((CH, 128), _f32),
            pltpu.VMEM((CH, 128), _f32),
            pltpu.VMEM((CH, 128), _f32),
            pltpu.SemaphoreType.DMA,
            pltpu.SemaphoreType.DMA,
            pltpu.SemaphoreType.DMA,
            pltpu.SemaphoreType.DMA,
        ],
    )


def _edge12_body(hrows, srcf, dstf, aspk, adpk, mp, out, acc_sh, as_sh, ad_sh,
                 m_vec, srcbuf, dstbuf, asid, adid, hidx, asg, adg, rowbuf,
                 denbuf, gsem, gsem2, gsem3, ssem):
    c = lax.axis_index("c")
    s = lax.axis_index("s")
    r0 = s * RPT
    d0 = DEN0 + s * (NDEN // 16)

    @pl.when(s == 0)
    def _():
        pltpu.sync_copy(aspk.at[c], as_sh)
        pltpu.sync_copy(adpk.at[c], ad_sh)

    _zero_rows(rowbuf, CH)
    _zero_rows(denbuf, CH)
    for q in range(10):
        pltpu.sync_copy(rowbuf, acc_sh.at[pl.ds(r0 + q * CH, CH)])
    pltpu.sync_copy(rowbuf.at[pl.ds(0, 32)], acc_sh.at[pl.ds(d0, 32)])
    pltpu.sync_copy(mp.at[c], m_vec)
    plsc.subcore_barrier()

    c_n = c * N
    zero16 = jnp.zeros((16,), _f32)

    def chunk(k, _):
        base = s * EPT + k * CH
        pltpu.sync_copy(srcf.at[pl.ds(base, CH)], srcbuf)
        pltpu.sync_copy(dstf.at[pl.ds(base, CH)], dstbuf)

        def prep(t, _):
            sl = pl.ds(t * 16, 16)
            sv = srcbuf[sl]
            asid[sl] = lax.shift_right_logical(sv, 5)
            adid[sl] = lax.shift_right_logical(dstbuf[sl], 5)
            hidx[sl] = sv + c_n
            return 0

        lax.fori_loop(0, CH // 16, prep, 0)

        g1 = pltpu.async_copy(as_sh.at[asid], asg, gsem)
        g2 = pltpu.async_copy(ad_sh.at[adid], adg, gsem2)
        g3 = pltpu.async_copy(hrows.at[hidx], rowbuf, gsem3)
        g1.wait()
        g2.wait()
        g3.wait()

        def sg(t, _):
            rbase = t * 16
            sl = pl.ds(rbase, 16)
            ridx = _iota16() + rbase
            lsrc = (srcbuf[sl] & 31) * 4
            lanes = (dstbuf[sl] & 31) * 4
            wv4 = []
            for h in range(4):
                asv = plsc.load_gather(asg, [ridx, lsrc + h])
                adv = plsc.load_gather(adg, [ridx, lanes + h])
                e = _leaky(asv + adv) - m_vec[h, pl.ds(0, 16)]
                wv4.append(jnp.exp(e))
            for h in range(4):
                plsc.store_scatter(denbuf, [ridx, lanes + h], wv4[h])
            for j in range(16):
                jj = rbase + j
                for h in range(4):
                    w = wv4[h][j]
                    for q in range(2):
                        sl2 = pl.ds(h * 32 + q * 16, 16)
                        rowbuf[jj, sl2] = rowbuf[jj, sl2] * w
            adid[sl] = adid[sl] + DEN0
            return 0

        lax.fori_loop(0, CH // 16, sg, 0)
        pltpu.sync_copy(rowbuf, acc_sh.at[dstbuf], add=True)
        pltpu.sync_copy(denbuf, acc_sh.at[adid], add=True)

        def zg(t, _):
            sl = pl.ds(t * 16, 16)
            ridx = _iota16() + t * 16
            lanes = (dstbuf[sl] & 31) * 4
            for h in range(4):
                plsc.store_scatter(denbuf, [ridx, lanes + h], zero16)
            return 0

        lax.fori_loop(0, CH // 16, zg, 0)
        return 0

    lax.fori_loop(0, EPT // CH, chunk, 0)
    plsc.subcore_barrier()
    for q in range(10):
        pltpu.sync_copy(acc_sh.at[pl.ds(r0 + q * CH, CH)], rowbuf)
        pltpu.sync_copy(rowbuf, out.at[c, pl.ds(r0 + q * CH, CH)])
    pltpu.sync_copy(acc_sh.at[pl.ds(d0, 32)], rowbuf.at[pl.ds(0, 32)])
    pltpu.sync_copy(rowbuf.at[pl.ds(0, 32)], out.at[c, pl.ds(d0, 32)])


@functools.cache
def _edge3_kernel():
    return pl.kernel(
        _edge3_body,
        out_type=jax.ShapeDtypeStruct((2, NPAD, 128), _f32),
        mesh=plsc.VectorSubcoreMesh(core_axis_name="c", subcore_axis_name="s"),
        compiler_params=pltpu.CompilerParams(needs_layout_passes=False),
        scratch_types=[
            pltpu.VMEM_SHARED((NPAD, 128), _f32),
            pltpu.VMEM((N,), _f32),
            pltpu.VMEM((N,), _f32),
            pltpu.VMEM((8, 128), _f32),
            pltpu.VMEM((CH,), jnp.int32),
            pltpu.VMEM((CH,), jnp.int32),
            pltpu.VMEM((CH, 128), _f32),
            pltpu.VMEM((1, CH), _f32),
            pltpu.SemaphoreType.DMA,
        ],
    )


def _edge3_body(hrows, srcf, dstf, ast, adt, mp, out, acc_sh, as_tab, ad_tab,
                m_vec, srcbuf, dstbuf, rowbuf, wtmp, sem):
    c = lax.axis_index("c")
    s = lax.axis_index("s")
    r0 = s * RPT

    _zero_rows(rowbuf, CH)
    for q in range(10):
        pltpu.sync_copy(rowbuf, acc_sh.at[pl.ds(r0 + q * CH, CH)])
    pltpu.sync_copy(ast, as_tab)
    pltpu.sync_copy(adt, ad_tab)
    pltpu.sync_copy(mp.at[0], m_vec)
    plsc.subcore_barrier()

    oh0 = (_iota16() == 0).astype(_f32)
    wid = s * 2 + c

    def chunk(k, _):
        base = wid * EPT3 + k * CH
        pltpu.sync_copy(srcf.at[pl.ds(base, CH)], srcbuf)
        pltpu.sync_copy(dstf.at[pl.ds(base, CH)], dstbuf)

        def wgroup(t, _):
            sl = pl.ds(t * 16, 16)
            asv = plsc.load_gather(as_tab, [srcbuf[sl]])
            adv = plsc.load_gather(ad_tab, [dstbuf[sl]])
            e = _leaky(asv + adv) - m_vec[0, pl.ds(0, 16)]
            wtmp[0, sl] = jnp.exp(e)
            return 0

        lax.fori_loop(0, CH // 16, wgroup, 0)
        pltpu.async_copy(hrows.at[srcbuf], rowbuf, sem).wait()

        def sgroup(t, _):
            rbase = t * 16
            wvec = wtmp[0, pl.ds(rbase, 16)]
            for j in range(16):
                jj = rbase + j
                w = wvec[j]
                for q in range(2):
                    sl2 = pl.ds(q * 16, 16)
                    rowbuf[jj, sl2] = rowbuf[jj, sl2] * w
                rowbuf[jj, pl.ds(32, 16)] = w * oh0
            return 0

        lax.fori_loop(0, CH // 16, sgroup, 0)
        pltpu.sync_copy(rowbuf, acc_sh.at[dstbuf], add=True)
        return 0

    lax.fori_loop(0, EPT3 // CH, chunk, 0)
    plsc.subcore_barrier()
    for q in range(10):
        pltpu.sync_copy(acc_sh.at[pl.ds(r0 + q * CH, CH)], rowbuf)
        pltpu.sync_copy(rowbuf, out.at[c, pl.ds(r0 + q * CH, CH)])


def _fold(W, a_s, a_d, heads):
    din = W.shape[0]
    wr = W.reshape(din, heads, HID)
    ws = jnp.einsum("dhc,hc->dh", wr, a_s)
    wd = jnp.einsum("dhc,hc->dh", wr, a_d)
    return ws, wd


def _den_view(acc):
    d = acc[:, DEN0:DEN0 + 320, :]
    return d.reshape(2, NPAD, 4)[:, :N, :]


def _pack_alpha(a):
    return jnp.pad(a, ((0, 0), (0, NPAD - N), (0, 0))).reshape(2, 320, 128)


def kernel(x, edge_index, batch, W1, a_src1, a_dst1, b1, W2, a_src2, a_dst2,
           b2, W3, a_src3, a_dst3, b3, Wl, bl):
    npad_e = EPAD - E
    src = jnp.concatenate([edge_index[0],
                           jnp.zeros((npad_e,), jnp.int32)])
    dst = jnp.concatenate([edge_index[1],
                           jnp.full((npad_e,), DUMMY_DST, jnp.int32)])

    ws1, wd1 = _fold(W1, a_src1, a_dst1, HEADS)
    ws2, wd2 = _fold(W2, a_src2, a_dst2, HEADS)
    ws3, wd3 = _fold(W3, a_src3, a_dst3, 1)
    w1h = jnp.stack([W1[:, 0:128], W1[:, 128:256]])
    w2h = jnp.stack([W2[:, 0:128], W2[:, 128:256]])
    wsd1 = jnp.stack([jnp.concatenate([ws1[:, 0:4], wd1[:, 0:4]], axis=1),
                      jnp.concatenate([ws1[:, 4:8], wd1[:, 4:8]], axis=1)])
    wsd2 = jnp.stack([jnp.concatenate([ws2[:, 0:4], wd2[:, 0:4]], axis=1),
                      jnp.concatenate([ws2[:, 4:8], wd2[:, 4:8]], axis=1)])
    w3a = jnp.concatenate([W3, ws3, jnp.zeros((W3.shape[0], 95), _f32)],
                          axis=1)

    h1, as1, ad1, m1 = _a1_call(x, w1h, wsd1)
    acc1 = _edge12_kernel()(h1, src, dst, _pack_alpha(as1),
                            _pack_alpha(ad1), m1)
    h2, as2, ad2, m2 = _a2_call(acc1, _den_view(acc1),
                                h1.reshape(2, N, 128), as1, ad1, m1,
                                b1.reshape(2, 128), w2h, wsd2)
    acc2 = _edge12_kernel()(h2, src, dst, _pack_alpha(as2),
                            _pack_alpha(ad2), m2)
    h3, ad3, m3 = _a3_call(acc2, _den_view(acc2),
                           h2.reshape(2, N, 128), as2, ad2, m2,
                           b2.reshape(2, 128), w3a, wd3)
    as3 = h3[:, 32:33]
    acc3 = _edge3_kernel()(h3, src, dst, as3.reshape(N), ad3.reshape(N), m3)
    out = _pool_call(acc3, h3, ad3, m3, b3.reshape(1, 32),
                     batch.reshape(N, 1), Wl, bl.reshape(1, OUT))
    return out
